# baseline (speedup 1.0000x reference)
import functools

import jax
import jax.numpy as jnp
from jax import lax
from jax.experimental import pallas as pl
from jax.experimental.pallas import tpu as pltpu
from jax.experimental.pallas import tpu_sc as plsc

_SEQ_LEN = 8192
_OUT_DIM = 1024
_NC = 2
_ROWS_PER_C = _SEQ_LEN // _NC

_CHUNK = 512
_NCHUNKS = _ROWS_PER_C // _CHUNK
_NBUF = 3


def _make_sc_copy():
    mesh = plsc.ScalarSubcoreMesh(axis_name="c", num_cores=_NC)

    @functools.partial(
        pl.kernel,
        mesh=mesh,
        out_type=jax.ShapeDtypeStruct((_SEQ_LEN, _OUT_DIM), jnp.float32),
        scratch_types=[
            pltpu.VMEM_SHARED((_NBUF, _CHUNK, _OUT_DIM), jnp.float32),
            pltpu.SemaphoreType.DMA,
            pltpu.SemaphoreType.DMA,
        ],
    )
    def copy_k(table_hbm, out_hbm, buf, gsem, ssem):
        base = lax.axis_index("c") * _ROWS_PER_C

        def gather(i):
            return pltpu.async_copy(
                table_hbm.at[pl.ds(base + i * _CHUNK, _CHUNK)],
                buf.at[i % _NBUF],
                gsem,
            )

        def scatter(i):
            return pltpu.async_copy(
                buf.at[i % _NBUF],
                out_hbm.at[pl.ds(base + i * _CHUNK, _CHUNK)],
                ssem,
            )

        gathers = [None] * _NCHUNKS
        scatters = [None] * _NCHUNKS
        for i in range(_NCHUNKS + 1):
            if i < _NCHUNKS:
                if i >= _NBUF:
                    scatters[i - _NBUF].wait()
                gathers[i] = gather(i)
            if i >= 1:
                gathers[i - 1].wait()
                scatters[i - 1] = scatter(i - 1)
        for i in range(max(0, _NCHUNKS - _NBUF), _NCHUNKS):
            scatters[i].wait()

    return copy_k


_sc_copy = _make_sc_copy()


@jax.jit
def kernel(inputs, pos_table):
    del inputs
    return _sc_copy(pos_table)


# speedup vs baseline: 1.0468x; 1.0468x over previous
import functools

import jax
import jax.numpy as jnp
from jax import lax
from jax.experimental import pallas as pl
from jax.experimental.pallas import tpu as pltpu
from jax.experimental.pallas import tpu_sc as plsc
from jax._src.pallas import mpmd as plmpmd

_SEQ_LEN = 8192
_OUT_DIM = 1024
_NC = 2
_NS = 16

_TEC_ROWS = 4096
_ROWS_PER_W = _TEC_ROWS // (_NC * _NS)
_T_CHUNKS = [32, 32, 32, 32]
_T_OFFS = [0, 32, 64, 96]
_T_BUF_ROWS = 32
_T_NBUF = 2

_SCS_ROWS = _SEQ_LEN - _TEC_ROWS
_ROWS_PER_C = _SCS_ROWS // _NC
_S_CHUNKS = [504, 504, 504, 504, 32]
_S_OFFS = [0, 504, 1008, 1512, 2016]
_S_BUF_ROWS = 504
_S_NBUF = 2


def _ring_copy(table_hbm, out_hbm, buf, gsem, ssem, base, chunks, offs, nbuf):
    def gather(i):
        return pltpu.async_copy(
            table_hbm.at[pl.ds(base + offs[i], chunks[i])],
            buf.at[i % nbuf, pl.ds(0, chunks[i])],
            gsem,
        )

    def scatter(i):
        return pltpu.async_copy(
            buf.at[i % nbuf, pl.ds(0, chunks[i])],
            out_hbm.at[pl.ds(base + offs[i], chunks[i])],
            ssem,
        )

    n = len(chunks)
    gathers = [None] * n
    scatters = [None] * n
    for i in range(n + 1):
        if i < n:
            if i >= nbuf:
                scatters[i - nbuf].wait()
            gathers[i] = gather(i)
        if i >= 1:
            gathers[i - 1].wait()
            scatters[i - 1] = scatter(i - 1)
    for i in range(max(0, n - nbuf), n):
        scatters[i].wait()


def _make_sc_copy():
    vec_mesh = plsc.VectorSubcoreMesh(core_axis_name="c", subcore_axis_name="s")
    scs_mesh = plsc.ScalarSubcoreMesh(axis_name="c", num_cores=_NC)

    def tec_fn(table_hbm, out_hbm, tbuf, sbuf, tg, ts, sg, ss):
        del sbuf, sg, ss
        wid = lax.axis_index("s") * _NC + lax.axis_index("c")
        base = wid * _ROWS_PER_W
        _ring_copy(table_hbm, out_hbm, tbuf, tg, ts, base, _T_CHUNKS, _T_OFFS,
                   _T_NBUF)

    def scs_fn(table_hbm, out_hbm, tbuf, sbuf, tg, ts, sg, ss):
        del tbuf, tg, ts
        base = _TEC_ROWS + lax.axis_index("c") * _ROWS_PER_C
        _ring_copy(table_hbm, out_hbm, sbuf, sg, ss, base, _S_CHUNKS, _S_OFFS,
                   _S_NBUF)

    vmem = pltpu.MemorySpace.VMEM @ vec_mesh
    return plmpmd.mpmd_map(
        [(scs_mesh, scs_fn), (vec_mesh, tec_fn)],
        out_types=jax.ShapeDtypeStruct((_SEQ_LEN, _OUT_DIM), jnp.float32),
        scratch_types=[
            vmem((_T_NBUF, _T_BUF_ROWS, _OUT_DIM), jnp.float32),
            pltpu.VMEM_SHARED((_S_NBUF, _S_BUF_ROWS, _OUT_DIM), jnp.float32),
            pltpu.SemaphoreType.DMA @ vec_mesh,
            pltpu.SemaphoreType.DMA @ vec_mesh,
            pltpu.SemaphoreType.DMA @ scs_mesh,
            pltpu.SemaphoreType.DMA @ scs_mesh,
        ],
    )


_sc_copy = _make_sc_copy()


@jax.jit
def kernel(inputs, pos_table):
    del inputs
    return _sc_copy(pos_table)
